# Initial kernel scaffold; baseline (speedup 1.0000x reference)
import jax
import jax.numpy as jnp
from jax.experimental import pallas as pl


def kernel(x, edge_index, batch, W1l, b1l, W1r, g1, be1, W2l, b2l, W2r, g2, be2, Wlin, blin):
    raise NotImplementedError("write your pallas kernel here")


# speedup vs baseline: 5.4785x; 5.4785x over previous
import functools

import jax
import jax.numpy as jnp
from jax import lax
from jax.experimental import pallas as pl
from jax.experimental.pallas import tpu as pltpu
from jax.experimental.pallas import tpu_sc as plsc

N = 10000
E = 320000
D = 128
G = 64
NC = 2
NS = 16
NW = NC * NS
EPW = E // NW
K = 80
NCHUNK = EPW // K
CP0 = 632
CP1 = N - 15 * CP0


def _slab_init(zsrc, accref, sid):
    @pl.when(sid < 15)
    def _():
        pltpu.sync_copy(zsrc, accref.at[pl.ds(sid * CP0, CP0)])

    @pl.when(sid == 15)
    def _():
        pltpu.sync_copy(zsrc.at[pl.ds(0, CP1)], accref.at[pl.ds(15 * CP0, CP1)])


def _slab_out(accref, outref, cid, sid):
    @pl.when(sid < 15)
    def _():
        pltpu.sync_copy(accref.at[pl.ds(sid * CP0, CP0)],
                        outref.at[pl.ds(cid * N + sid * CP0, CP0)])

    @pl.when(sid == 15)
    def _():
        pltpu.sync_copy(accref.at[pl.ds(15 * CP0, CP1)],
                        outref.at[pl.ds(cid * N + 15 * CP0, CP1)])


def _sc_agg(with_counts):
    mesh = plsc.VectorSubcoreMesh(core_axis_name="c", subcore_axis_name="s")
    out_type = [jax.ShapeDtypeStruct((NC * N, D), jnp.float32)]
    scratch = [
        pltpu.VMEM((K,), jnp.int32),
        pltpu.VMEM((K,), jnp.int32),
        pltpu.VMEM((K, D), jnp.float32),
        pltpu.VMEM_SHARED((N, D), jnp.float32),
        pltpu.SemaphoreType.DMA,
    ]
    if with_counts:
        out_type.append(jax.ShapeDtypeStruct((NC * N,), jnp.float32))
        scratch += [
            pltpu.VMEM((K,), jnp.float32),
            pltpu.VMEM_SHARED((N,), jnp.float32),
            pltpu.VMEM((CP0,), jnp.float32),
        ]

    def body_counts(table, src, dst, zrow, zcnt, ones,
                    out_acc, out_cnt, sidx, didx, rows, acc, sem, ones_v, cacc,
                    cnt_v):
        cid = lax.axis_index("c")
        sid = lax.axis_index("s")
        _slab_init(zrow, acc, sid)
        pltpu.sync_copy(zcnt, cnt_v)

        @pl.when(sid < 15)
        def _():
            pltpu.sync_copy(cnt_v, cacc.at[pl.ds(sid * CP0, CP0)])

        @pl.when(sid == 15)
        def _():
            pltpu.sync_copy(cnt_v.at[pl.ds(0, CP1)],
                            cacc.at[pl.ds(15 * CP0, CP1)])

        pltpu.sync_copy(ones, ones_v)
        plsc.subcore_barrier()
        base = (cid * NS + sid) * EPW

        def step(i, carry):
            off = base + i * K
            pltpu.sync_copy(src.at[pl.ds(off, K)], sidx)
            pltpu.sync_copy(dst.at[pl.ds(off, K)], didx)
            pltpu.async_copy(table.at[sidx], rows, sem).wait()
            pltpu.sync_copy(rows, acc.at[didx], add=True)
            pltpu.sync_copy(ones_v, cacc.at[didx], add=True)
            return carry

        lax.fori_loop(0, NCHUNK, step, 0)
        plsc.subcore_barrier()
        _slab_out(acc, out_acc, cid, sid)

        @pl.when(sid < 15)
        def _():
            pltpu.sync_copy(cacc.at[pl.ds(sid * CP0, CP0)], cnt_v)
            pltpu.sync_copy(cnt_v,
                            out_cnt.at[pl.ds(cid * N + sid * CP0, CP0)])

        @pl.when(sid == 15)
        def _():
            pltpu.sync_copy(cacc.at[pl.ds(15 * CP0, CP1)],
                            cnt_v.at[pl.ds(0, CP1)])
            pltpu.sync_copy(cnt_v.at[pl.ds(0, CP1)],
                            out_cnt.at[pl.ds(cid * N + 15 * CP0, CP1)])

    def body_plain(table, src, dst, zrow,
                   out_acc, sidx, didx, rows, acc, sem):
        cid = lax.axis_index("c")
        sid = lax.axis_index("s")
        _slab_init(zrow, acc, sid)
        plsc.subcore_barrier()
        base = (cid * NS + sid) * EPW

        def step(i, carry):
            off = base + i * K
            pltpu.sync_copy(src.at[pl.ds(off, K)], sidx)
            pltpu.sync_copy(dst.at[pl.ds(off, K)], didx)
            pltpu.async_copy(table.at[sidx], rows, sem).wait()
            pltpu.sync_copy(rows, acc.at[didx], add=True)
            return carry

        lax.fori_loop(0, NCHUNK, step, 0)
        plsc.subcore_barrier()
        _slab_out(acc, out_acc, cid, sid)

    body = body_counts if with_counts else body_plain
    return pl.kernel(body, mesh=mesh, out_type=out_type, scratch_types=scratch)


def _tc_pre(x, w1l, w1r, b1l):
    def body(x_ref, wl_ref, wr_ref, b_ref, t_ref, p_ref):
        xv = x_ref[...]
        t_ref[...] = jnp.dot(xv, wl_ref[...], preferred_element_type=jnp.float32)
        p_ref[...] = (jnp.dot(xv, wr_ref[...], preferred_element_type=jnp.float32)
                      + b_ref[...])

    return pl.pallas_call(
        body,
        out_shape=[jax.ShapeDtypeStruct((N, D), jnp.float32),
                   jax.ShapeDtypeStruct((N, D), jnp.float32)],
    )(x, w1l, w1r, b1l)


def _tc_mid(acc2n, cnt_a, cnt_b, p1, g1, be1, w2l, w2r, b2l):
    def body(a_ref, ca_ref, cb_ref, p_ref, g_ref, be_ref, wl_ref, wr_ref,
             b_ref, t_ref, p2_ref, cnt_ref):
        agg = a_ref[0:N, :] + a_ref[N:2 * N, :]
        cnt = jnp.maximum(ca_ref[...] + cb_ref[...], 1.0)
        s = agg / cnt + p_ref[...]
        mu = jnp.mean(s, axis=0, keepdims=True)
        var = jnp.mean((s - mu) ** 2, axis=0, keepdims=True)
        h = (s - mu) * lax.rsqrt(var + 1e-5) * g_ref[...] + be_ref[...]
        h = jnp.maximum(h, 0.0)
        t_ref[...] = jnp.dot(h, wl_ref[...], preferred_element_type=jnp.float32)
        p2_ref[...] = (jnp.dot(h, wr_ref[...], preferred_element_type=jnp.float32)
                       + b_ref[...])
        cnt_ref[...] = cnt

    return pl.pallas_call(
        body,
        out_shape=[jax.ShapeDtypeStruct((N, D), jnp.float32),
                   jax.ShapeDtypeStruct((N, D), jnp.float32),
                   jax.ShapeDtypeStruct((N, 1), jnp.float32)],
    )(acc2n, cnt_a, cnt_b, p1, g1, be1, w2l, w2r, b2l)


def _tc_post(acc2n, cnt, p2, g2, be2, batch2d, wlin_pad, blin_pad):
    def body(a_ref, c_ref, p_ref, g_ref, be_ref, bt_ref, wl_ref, bl_ref, o_ref):
        agg = a_ref[0:N, :] + a_ref[N:2 * N, :]
        s = agg / c_ref[...] + p_ref[...]
        mu = jnp.mean(s, axis=0, keepdims=True)
        var = jnp.mean((s - mu) ** 2, axis=0, keepdims=True)
        h = (s - mu) * lax.rsqrt(var + 1e-5) * g_ref[...] + be_ref[...]
        h = jnp.maximum(h, 0.0)
        gid = lax.broadcasted_iota(jnp.int32, (G, N), 0)
        oh = (gid == bt_ref[...]).astype(jnp.float32)
        sums = jnp.dot(oh, h, preferred_element_type=jnp.float32)
        cnts = jnp.maximum(jnp.sum(oh, axis=1, keepdims=True), 1.0)
        pooled = sums / cnts
        z = jnp.dot(pooled, wl_ref[...], preferred_element_type=jnp.float32) + bl_ref[...]
        o_ref[...] = 1.0 / (1.0 + jnp.exp(-z))

    return pl.pallas_call(
        body,
        out_shape=jax.ShapeDtypeStruct((G, D), jnp.float32),
    )(acc2n, cnt, p2, g2, be2, batch2d, wlin_pad, blin_pad)


def kernel(x, edge_index, batch, W1l, b1l, W1r, g1, be1, W2l, b2l, W2r, g2,
           be2, Wlin, blin):
    src = edge_index[0]
    dst = edge_index[1]
    zrow = jnp.zeros((CP0, D), jnp.float32)
    zcnt = jnp.zeros((CP0,), jnp.float32)
    ones = jnp.ones((K,), jnp.float32)

    t1, p1 = _tc_pre(x, W1l, W1r, b1l.reshape(1, D))

    acc1, cntp = _sc_agg(True)(t1, src, dst, zrow, zcnt, ones)

    t2, p2, cnt = _tc_mid(acc1, cntp[:N].reshape(N, 1),
                          cntp[N:].reshape(N, 1),
                          p1, g1.reshape(1, D), be1.reshape(1, D),
                          W2l, W2r, b2l.reshape(1, D))

    (acc2,) = _sc_agg(False)(t2, src, dst, zrow)

    wlin_pad = jnp.zeros((D, D), jnp.float32).at[:, :Wlin.shape[1]].set(Wlin)
    blin_pad = jnp.zeros((1, D), jnp.float32).at[0, :blin.shape[0]].set(blin)
    out = _tc_post(acc2, cnt, p2,
                   g2.reshape(1, D), be2.reshape(1, D),
                   batch.reshape(1, N).astype(jnp.int32), wlin_pad, blin_pad)
    return out[:, :Wlin.shape[1]]


# speedup vs baseline: 10.1418x; 1.8512x over previous
import functools

import jax
import jax.numpy as jnp
from jax import lax
from jax.experimental import pallas as pl
from jax.experimental.pallas import tpu as pltpu
from jax.experimental.pallas import tpu_sc as plsc

N = 10000
E = 320000
D = 128
G = 64
NC = 2
NS = 16
NW = NC * NS
EPW = E // NW
K = 80
NCHUNK = EPW // K
CP0 = 632
CP1 = N - 15 * CP0


def _slab_init(zsrc, accref, sid):
    @pl.when(sid < 15)
    def _():
        pltpu.sync_copy(zsrc, accref.at[pl.ds(sid * CP0, CP0)])

    @pl.when(sid == 15)
    def _():
        pltpu.sync_copy(zsrc.at[pl.ds(0, CP1)], accref.at[pl.ds(15 * CP0, CP1)])


def _slab_out(accref, outref, cid, sid):
    @pl.when(sid < 15)
    def _():
        pltpu.sync_copy(accref.at[pl.ds(sid * CP0, CP0)],
                        outref.at[pl.ds(cid * N + sid * CP0, CP0)])

    @pl.when(sid == 15)
    def _():
        pltpu.sync_copy(accref.at[pl.ds(15 * CP0, CP1)],
                        outref.at[pl.ds(cid * N + 15 * CP0, CP1)])


def _sc_agg(with_counts):
    mesh = plsc.VectorSubcoreMesh(core_axis_name="c", subcore_axis_name="s")
    out_type = [jax.ShapeDtypeStruct((NC * N, D), jnp.float32)]
    scratch = (
        [pltpu.VMEM((2, K), jnp.int32)] * 2
        + [pltpu.VMEM((2, K), jnp.int32)] * 0
        + [pltpu.VMEM((K, D), jnp.float32) for _ in range(2)]
        + [pltpu.SemaphoreType.DMA] * 4
        + [pltpu.VMEM_SHARED((N, D), jnp.float32)]
    )
    if with_counts:
        out_type.append(jax.ShapeDtypeStruct((NC * N,), jnp.float32))
        scratch += [
            pltpu.VMEM((K,), jnp.float32),
            pltpu.VMEM_SHARED((N,), jnp.float32),
            pltpu.VMEM((CP0,), jnp.float32),
        ]

    def body(table, src, dst, zrow, *rest):
        if with_counts:
            zcnt, ones, out_acc, out_cnt = rest[0], rest[1], rest[2], rest[3]
            sc = rest[4:]
        else:
            out_acc = rest[0]
            sc = rest[1:]
        iset = sc[0:2]
        rows = sc[2:4]
        semg = sc[4:6]
        semi = sc[6:8]
        acc = sc[8]
        if with_counts:
            ones_v, cacc, cnt_v = sc[9:12]

        cid = lax.axis_index("c")
        sid = lax.axis_index("s")
        base = (cid * NS + sid) * EPW

        def idx_issue(c, s):
            off = base + c * K
            pltpu.async_copy(src.at[pl.ds(off, K)], iset[s].at[0], semi[s])
            pltpu.async_copy(dst.at[pl.ds(off, K)], iset[s].at[1], semi[s])

        def idx_wait(s):
            pltpu.make_async_copy(src.at[pl.ds(0, K)], iset[s].at[0],
                                  semi[s]).wait()
            pltpu.make_async_copy(dst.at[pl.ds(0, K)], iset[s].at[1],
                                  semi[s]).wait()

        def gather(s):
            return pltpu.async_copy(table.at[iset[s].at[0]], rows[s], semg[s])

        def scatter(s):
            pltpu.sync_copy(rows[s], acc.at[iset[s].at[1]], add=True)
            if with_counts:
                pltpu.sync_copy(ones_v, cacc.at[iset[s].at[1]], add=True)

        idx_issue(0, 0)
        _slab_init(zrow, acc, sid)
        if with_counts:
            pltpu.sync_copy(zcnt, cnt_v)

            @pl.when(sid < 15)
            def _():
                pltpu.sync_copy(cnt_v, cacc.at[pl.ds(sid * CP0, CP0)])

            @pl.when(sid == 15)
            def _():
                pltpu.sync_copy(cnt_v.at[pl.ds(0, CP1)],
                                cacc.at[pl.ds(15 * CP0, CP1)])

            pltpu.sync_copy(ones, ones_v)
        plsc.subcore_barrier()

        def pair(j, carry):
            a = 2 * j
            idx_wait(0)
            h0 = gather(0)
            idx_issue(a + 1, 1)
            idx_wait(1)
            h1 = gather(1)
            h0.wait()
            scatter(0)
            idx_issue(a + 2, 0)
            h1.wait()
            scatter(1)
            return carry

        lax.fori_loop(0, (NCHUNK - 1) // 2, pair, 0)
        idx_wait(0)
        he = gather(0)
        he.wait()
        scatter(0)
        plsc.subcore_barrier()
        _slab_out(acc, out_acc, cid, sid)
        if with_counts:
            @pl.when(sid < 15)
            def _():
                pltpu.sync_copy(cacc.at[pl.ds(sid * CP0, CP0)], cnt_v)
                pltpu.sync_copy(cnt_v,
                                out_cnt.at[pl.ds(cid * N + sid * CP0, CP0)])

            @pl.when(sid == 15)
            def _():
                pltpu.sync_copy(cacc.at[pl.ds(15 * CP0, CP1)],
                                cnt_v.at[pl.ds(0, CP1)])
                pltpu.sync_copy(cnt_v.at[pl.ds(0, CP1)],
                                out_cnt.at[pl.ds(cid * N + 15 * CP0, CP1)])

    return pl.kernel(body, mesh=mesh, out_type=out_type, scratch_types=scratch)


def _tc_pre(x, w1l, w1r, b1l):
    def body(x_ref, wl_ref, wr_ref, b_ref, t_ref, p_ref):
        xv = x_ref[...]
        t_ref[...] = jnp.dot(xv, wl_ref[...], preferred_element_type=jnp.float32)
        p_ref[...] = (jnp.dot(xv, wr_ref[...], preferred_element_type=jnp.float32)
                      + b_ref[...])

    return pl.pallas_call(
        body,
        out_shape=[jax.ShapeDtypeStruct((N, D), jnp.float32),
                   jax.ShapeDtypeStruct((N, D), jnp.float32)],
    )(x, w1l, w1r, b1l)


def _tc_mid(acc2n, cnt_a, cnt_b, p1, g1, be1, w2l, w2r, b2l):
    def body(a_ref, ca_ref, cb_ref, p_ref, g_ref, be_ref, wl_ref, wr_ref,
             b_ref, t_ref, p2_ref, cnt_ref):
        agg = a_ref[0:N, :] + a_ref[N:2 * N, :]
        cnt = jnp.maximum(ca_ref[...] + cb_ref[...], 1.0)
        s = agg / cnt + p_ref[...]
        mu = jnp.mean(s, axis=0, keepdims=True)
        var = jnp.mean((s - mu) ** 2, axis=0, keepdims=True)
        h = (s - mu) * lax.rsqrt(var + 1e-5) * g_ref[...] + be_ref[...]
        h = jnp.maximum(h, 0.0)
        t_ref[...] = jnp.dot(h, wl_ref[...], preferred_element_type=jnp.float32)
        p2_ref[...] = (jnp.dot(h, wr_ref[...], preferred_element_type=jnp.float32)
                       + b_ref[...])
        cnt_ref[...] = cnt

    return pl.pallas_call(
        body,
        out_shape=[jax.ShapeDtypeStruct((N, D), jnp.float32),
                   jax.ShapeDtypeStruct((N, D), jnp.float32),
                   jax.ShapeDtypeStruct((N, 1), jnp.float32)],
    )(acc2n, cnt_a, cnt_b, p1, g1, be1, w2l, w2r, b2l)


def _tc_post(acc2n, cnt, p2, g2, be2, batch2d, wlin_pad, blin_pad):
    def body(a_ref, c_ref, p_ref, g_ref, be_ref, bt_ref, wl_ref, bl_ref, o_ref):
        agg = a_ref[0:N, :] + a_ref[N:2 * N, :]
        s = agg / c_ref[...] + p_ref[...]
        mu = jnp.mean(s, axis=0, keepdims=True)
        var = jnp.mean((s - mu) ** 2, axis=0, keepdims=True)
        h = (s - mu) * lax.rsqrt(var + 1e-5) * g_ref[...] + be_ref[...]
        h = jnp.maximum(h, 0.0)
        gid = lax.broadcasted_iota(jnp.int32, (G, N), 0)
        oh = (gid == bt_ref[...]).astype(jnp.float32)
        sums = jnp.dot(oh, h, preferred_element_type=jnp.float32)
        cnts = jnp.maximum(jnp.sum(oh, axis=1, keepdims=True), 1.0)
        pooled = sums / cnts
        z = jnp.dot(pooled, wl_ref[...], preferred_element_type=jnp.float32) + bl_ref[...]
        o_ref[...] = 1.0 / (1.0 + jnp.exp(-z))

    return pl.pallas_call(
        body,
        out_shape=jax.ShapeDtypeStruct((G, D), jnp.float32),
    )(acc2n, cnt, p2, g2, be2, batch2d, wlin_pad, blin_pad)


def kernel(x, edge_index, batch, W1l, b1l, W1r, g1, be1, W2l, b2l, W2r, g2,
           be2, Wlin, blin):
    src = edge_index[0]
    dst = edge_index[1]
    zrow = jnp.zeros((CP0, D), jnp.float32)
    zcnt = jnp.zeros((CP0,), jnp.float32)
    ones = jnp.ones((K,), jnp.float32)

    t1, p1 = _tc_pre(x, W1l, W1r, b1l.reshape(1, D))

    acc1, cntp = _sc_agg(True)(t1, src, dst, zrow, zcnt, ones)

    t2, p2, cnt = _tc_mid(acc1, cntp[:N].reshape(N, 1),
                          cntp[N:].reshape(N, 1),
                          p1, g1.reshape(1, D), be1.reshape(1, D),
                          W2l, W2r, b2l.reshape(1, D))

    (acc2,) = _sc_agg(False)(t2, src, dst, zrow)

    wlin_pad = jnp.zeros((D, D), jnp.float32).at[:, :Wlin.shape[1]].set(Wlin)
    blin_pad = jnp.zeros((1, D), jnp.float32).at[0, :blin.shape[0]].set(blin)
    out = _tc_post(acc2, cnt, p2,
                   g2.reshape(1, D), be2.reshape(1, D),
                   batch.reshape(1, N).astype(jnp.int32), wlin_pad, blin_pad)
    return out[:, :Wlin.shape[1]]


# speedup vs baseline: 10.4235x; 1.0278x over previous
import functools

import jax
import jax.numpy as jnp
from jax import lax
from jax.experimental import pallas as pl
from jax.experimental.pallas import tpu as pltpu
from jax.experimental.pallas import tpu_sc as plsc

N = 10000
E = 320000
D = 128
G = 64
NC = 2
NS = 16
NW = NC * NS
EPW = E // NW
K = 80
NCHUNK = EPW // K
CP0 = 632
CP1 = N - 15 * CP0


def _slab_init(zsrc, accref, sid):
    @pl.when(sid < 15)
    def _():
        pltpu.sync_copy(zsrc, accref.at[pl.ds(sid * CP0, CP0)])

    @pl.when(sid == 15)
    def _():
        pltpu.sync_copy(zsrc.at[pl.ds(0, CP1)], accref.at[pl.ds(15 * CP0, CP1)])


def _slab_out(accref, outref, cid, sid):
    @pl.when(sid < 15)
    def _():
        pltpu.sync_copy(accref.at[pl.ds(sid * CP0, CP0)],
                        outref.at[pl.ds(cid * N + sid * CP0, CP0)])

    @pl.when(sid == 15)
    def _():
        pltpu.sync_copy(accref.at[pl.ds(15 * CP0, CP1)],
                        outref.at[pl.ds(cid * N + 15 * CP0, CP1)])


def _sc_agg(with_counts):
    mesh = plsc.VectorSubcoreMesh(core_axis_name="c", subcore_axis_name="s")
    out_type = [jax.ShapeDtypeStruct((NC * N, D), jnp.float32)]
    scratch = (
        [pltpu.VMEM((2, K), jnp.int32)] * 4
        + [pltpu.VMEM((K, D), jnp.float32) for _ in range(2)]
        + [pltpu.SemaphoreType.DMA] * 6
        + [pltpu.VMEM_SHARED((N, D), jnp.float32)]
    )
    if with_counts:
        out_type.append(jax.ShapeDtypeStruct((NC * N,), jnp.float32))
        scratch += [
            pltpu.VMEM((K,), jnp.float32),
            pltpu.VMEM_SHARED((N,), jnp.float32),
            pltpu.VMEM((CP0,), jnp.float32),
        ]

    def body(table, src, dst, zrow, *rest):
        if with_counts:
            zcnt, ones, out_acc, out_cnt = rest[0], rest[1], rest[2], rest[3]
            sc = rest[4:]
        else:
            out_acc = rest[0]
            sc = rest[1:]
        iset = sc[0:4]
        rows = sc[4:6]
        semg = sc[6:8]
        semi = sc[8:12]
        acc = sc[12]
        if with_counts:
            ones_v, cacc, cnt_v = sc[13:16]

        cid = lax.axis_index("c")
        sid = lax.axis_index("s")
        base = (cid * NS + sid) * EPW

        def idx_issue(c, s):
            off = base + c * K
            pltpu.async_copy(src.at[pl.ds(off, K)], iset[s].at[0], semi[s])
            pltpu.async_copy(dst.at[pl.ds(off, K)], iset[s].at[1], semi[s])

        def idx_wait(s):
            pltpu.make_async_copy(src.at[pl.ds(0, K)], iset[s].at[0],
                                  semi[s]).wait()
            pltpu.make_async_copy(dst.at[pl.ds(0, K)], iset[s].at[1],
                                  semi[s]).wait()

        def gather(b, s):
            return pltpu.async_copy(table.at[iset[s].at[0]], rows[b], semg[b])

        def scatter(b, s):
            pltpu.sync_copy(rows[b], acc.at[iset[s].at[1]], add=True)
            if with_counts:
                pltpu.sync_copy(ones_v, cacc.at[iset[s].at[1]], add=True)

        def iss(c, s):
            @pl.when(c < NCHUNK)
            def _():
                idx_issue(c, s)

        for p in range(4):
            idx_issue(p, p)
        _slab_init(zrow, acc, sid)
        if with_counts:
            pltpu.sync_copy(zcnt, cnt_v)

            @pl.when(sid < 15)
            def _():
                pltpu.sync_copy(cnt_v, cacc.at[pl.ds(sid * CP0, CP0)])

            @pl.when(sid == 15)
            def _():
                pltpu.sync_copy(cnt_v.at[pl.ds(0, CP1)],
                                cacc.at[pl.ds(15 * CP0, CP1)])

            pltpu.sync_copy(ones, ones_v)
        plsc.subcore_barrier()

        def quad(j, carry):
            c = 4 * j
            idx_wait(0)
            h0 = gather(0, 0)
            idx_wait(1)
            h1 = gather(1, 1)
            h0.wait()
            scatter(0, 0)
            idx_wait(2)
            h2 = gather(0, 2)
            h1.wait()
            scatter(1, 1)
            iss(c + 4, 0)
            iss(c + 5, 1)
            idx_wait(3)
            h3 = gather(1, 3)
            h2.wait()
            scatter(0, 2)
            iss(c + 6, 2)
            h3.wait()
            scatter(1, 3)
            iss(c + 7, 3)
            return carry

        lax.fori_loop(0, NCHUNK // 4, quad, 0)
        idx_wait(0)
        he = gather(0, 0)
        he.wait()
        scatter(0, 0)
        plsc.subcore_barrier()
        _slab_out(acc, out_acc, cid, sid)
        if with_counts:
            @pl.when(sid < 15)
            def _():
                pltpu.sync_copy(cacc.at[pl.ds(sid * CP0, CP0)], cnt_v)
                pltpu.sync_copy(cnt_v,
                                out_cnt.at[pl.ds(cid * N + sid * CP0, CP0)])

            @pl.when(sid == 15)
            def _():
                pltpu.sync_copy(cacc.at[pl.ds(15 * CP0, CP1)],
                                cnt_v.at[pl.ds(0, CP1)])
                pltpu.sync_copy(cnt_v.at[pl.ds(0, CP1)],
                                out_cnt.at[pl.ds(cid * N + 15 * CP0, CP1)])

    return pl.kernel(body, mesh=mesh, out_type=out_type, scratch_types=scratch)


def _tc_pre(x, w1l, w1r, b1l):
    def body(x_ref, wl_ref, wr_ref, b_ref, t_ref, p_ref):
        xv = x_ref[...]
        t_ref[...] = jnp.dot(xv, wl_ref[...], preferred_element_type=jnp.float32)
        p_ref[...] = (jnp.dot(xv, wr_ref[...], preferred_element_type=jnp.float32)
                      + b_ref[...])

    return pl.pallas_call(
        body,
        out_shape=[jax.ShapeDtypeStruct((N, D), jnp.float32),
                   jax.ShapeDtypeStruct((N, D), jnp.float32)],
    )(x, w1l, w1r, b1l)


def _tc_mid(acc2n, cnt_a, cnt_b, p1, g1, be1, w2l, w2r, b2l):
    def body(a_ref, ca_ref, cb_ref, p_ref, g_ref, be_ref, wl_ref, wr_ref,
             b_ref, t_ref, p2_ref, cnt_ref):
        agg = a_ref[0:N, :] + a_ref[N:2 * N, :]
        cnt = jnp.maximum(ca_ref[...] + cb_ref[...], 1.0)
        s = agg / cnt + p_ref[...]
        mu = jnp.mean(s, axis=0, keepdims=True)
        var = jnp.mean((s - mu) ** 2, axis=0, keepdims=True)
        h = (s - mu) * lax.rsqrt(var + 1e-5) * g_ref[...] + be_ref[...]
        h = jnp.maximum(h, 0.0)
        t_ref[...] = jnp.dot(h, wl_ref[...], preferred_element_type=jnp.float32)
        p2_ref[...] = (jnp.dot(h, wr_ref[...], preferred_element_type=jnp.float32)
                       + b_ref[...])
        cnt_ref[...] = cnt

    return pl.pallas_call(
        body,
        out_shape=[jax.ShapeDtypeStruct((N, D), jnp.float32),
                   jax.ShapeDtypeStruct((N, D), jnp.float32),
                   jax.ShapeDtypeStruct((N, 1), jnp.float32)],
    )(acc2n, cnt_a, cnt_b, p1, g1, be1, w2l, w2r, b2l)


def _tc_post(acc2n, cnt, p2, g2, be2, batch2d, wlin_pad, blin_pad):
    def body(a_ref, c_ref, p_ref, g_ref, be_ref, bt_ref, wl_ref, bl_ref, o_ref):
        agg = a_ref[0:N, :] + a_ref[N:2 * N, :]
        s = agg / c_ref[...] + p_ref[...]
        mu = jnp.mean(s, axis=0, keepdims=True)
        var = jnp.mean((s - mu) ** 2, axis=0, keepdims=True)
        h = (s - mu) * lax.rsqrt(var + 1e-5) * g_ref[...] + be_ref[...]
        h = jnp.maximum(h, 0.0)
        gid = lax.broadcasted_iota(jnp.int32, (G, N), 0)
        oh = (gid == bt_ref[...]).astype(jnp.float32)
        sums = jnp.dot(oh, h, preferred_element_type=jnp.float32)
        cnts = jnp.maximum(jnp.sum(oh, axis=1, keepdims=True), 1.0)
        pooled = sums / cnts
        z = jnp.dot(pooled, wl_ref[...], preferred_element_type=jnp.float32) + bl_ref[...]
        o_ref[...] = 1.0 / (1.0 + jnp.exp(-z))

    return pl.pallas_call(
        body,
        out_shape=jax.ShapeDtypeStruct((G, D), jnp.float32),
    )(acc2n, cnt, p2, g2, be2, batch2d, wlin_pad, blin_pad)


def kernel(x, edge_index, batch, W1l, b1l, W1r, g1, be1, W2l, b2l, W2r, g2,
           be2, Wlin, blin):
    src = edge_index[0]
    dst = edge_index[1]
    zrow = jnp.zeros((CP0, D), jnp.float32)
    zcnt = jnp.zeros((CP0,), jnp.float32)
    ones = jnp.ones((K,), jnp.float32)

    t1, p1 = _tc_pre(x, W1l, W1r, b1l.reshape(1, D))

    acc1, cntp = _sc_agg(True)(t1, src, dst, zrow, zcnt, ones)

    t2, p2, cnt = _tc_mid(acc1, cntp[:N].reshape(N, 1),
                          cntp[N:].reshape(N, 1),
                          p1, g1.reshape(1, D), be1.reshape(1, D),
                          W2l, W2r, b2l.reshape(1, D))

    (acc2,) = _sc_agg(False)(t2, src, dst, zrow)

    wlin_pad = jnp.zeros((D, D), jnp.float32).at[:, :Wlin.shape[1]].set(Wlin)
    blin_pad = jnp.zeros((1, D), jnp.float32).at[0, :blin.shape[0]].set(blin)
    out = _tc_post(acc2, cnt, p2,
                   g2.reshape(1, D), be2.reshape(1, D),
                   batch.reshape(1, N).astype(jnp.int32), wlin_pad, blin_pad)
    return out[:, :Wlin.shape[1]]


# speedup vs baseline: 12.1067x; 1.1615x over previous
import functools

import jax
import jax.numpy as jnp
from jax import lax
from jax.experimental import pallas as pl
from jax.experimental.pallas import tpu as pltpu
from jax.experimental.pallas import tpu_sc as plsc

N = 10000
E = 320000
D = 128
G = 64
NC = 2
NS = 16
NW = NC * NS
EPW = E // NW
K = 80
NCHUNK = EPW // K
CP0 = 632
CP1 = N - 15 * CP0


def _slab_init(zsrc, accref, sid):
    @pl.when(sid < 15)
    def _():
        pltpu.sync_copy(zsrc, accref.at[pl.ds(sid * CP0, CP0)])

    @pl.when(sid == 15)
    def _():
        pltpu.sync_copy(zsrc.at[pl.ds(0, CP1)], accref.at[pl.ds(15 * CP0, CP1)])


def _slab_out(accref, outref, cid, sid):
    @pl.when(sid < 15)
    def _():
        pltpu.sync_copy(accref.at[pl.ds(sid * CP0, CP0)],
                        outref.at[pl.ds(cid * N + sid * CP0, CP0)])

    @pl.when(sid == 15)
    def _():
        pltpu.sync_copy(accref.at[pl.ds(15 * CP0, CP1)],
                        outref.at[pl.ds(cid * N + 15 * CP0, CP1)])


def _sc_agg(with_counts):
    mesh = plsc.VectorSubcoreMesh(core_axis_name="c", subcore_axis_name="s")
    out_type = [jax.ShapeDtypeStruct((NC * N, D), jnp.float32)]
    scratch = (
        [pltpu.VMEM((2, K), jnp.int32)] * 8
        + [pltpu.VMEM((K, D), jnp.float32) for _ in range(2)]
        + [pltpu.SemaphoreType.DMA] * 12
        + [pltpu.VMEM_SHARED((N, D), jnp.float32)]
    )
    if with_counts:
        out_type.append(jax.ShapeDtypeStruct((NC * N,), jnp.float32))
        scratch += [
            pltpu.VMEM((K,), jnp.float32),
            pltpu.VMEM_SHARED((N,), jnp.float32),
            pltpu.VMEM((CP0,), jnp.float32),
        ]

    def body(table, src, dst, zrow, *rest):
        if with_counts:
            zcnt, ones, out_acc, out_cnt = rest[0], rest[1], rest[2], rest[3]
            sc = rest[4:]
        else:
            out_acc = rest[0]
            sc = rest[1:]
        iset = sc[0:8]
        rows = sc[8:10]
        semg = sc[10:12]
        semsc = sc[12:14]
        semi = sc[14:22]
        acc = sc[22]
        if with_counts:
            ones_v, cacc, cnt_v = sc[23:26]

        cid = lax.axis_index("c")
        sid = lax.axis_index("s")
        base = (cid * NS + sid) * EPW

        def idx_issue(c, s):
            off = base + c * K
            pltpu.async_copy(src.at[pl.ds(off, K)], iset[s].at[0], semi[s])
            pltpu.async_copy(dst.at[pl.ds(off, K)], iset[s].at[1], semi[s])

        def idx_wait(s):
            pltpu.make_async_copy(src.at[pl.ds(0, K)], iset[s].at[0],
                                  semi[s]).wait()
            pltpu.make_async_copy(dst.at[pl.ds(0, K)], iset[s].at[1],
                                  semi[s]).wait()

        def gissue(b, s):
            pltpu.async_copy(table.at[iset[s].at[0]], rows[b], semg[b])

        def gwait(b, s):
            pltpu.make_async_copy(table.at[iset[s].at[0]], rows[b],
                                  semg[b]).wait()

        def scatter_async(b, s):
            pltpu.async_copy(rows[b], acc.at[iset[s].at[1]], semsc[b],
                             add=True)
            if with_counts:
                pltpu.async_copy(ones_v, cacc.at[iset[s].at[1]], semsc[b],
                                 add=True)

        def scw(b, s):
            pltpu.make_async_copy(rows[b], acc.at[iset[s].at[1]],
                                  semsc[b]).wait()
            if with_counts:
                pltpu.make_async_copy(ones_v, cacc.at[iset[s].at[1]],
                                      semsc[b]).wait()

        for p in range(5):
            idx_issue(p, p)
        _slab_init(zrow, acc, sid)
        if with_counts:
            pltpu.sync_copy(zcnt, cnt_v)

            @pl.when(sid < 15)
            def _():
                pltpu.sync_copy(cnt_v, cacc.at[pl.ds(sid * CP0, CP0)])

            @pl.when(sid == 15)
            def _():
                pltpu.sync_copy(cnt_v.at[pl.ds(0, CP1)],
                                cacc.at[pl.ds(15 * CP0, CP1)])

            pltpu.sync_copy(ones, ones_v)
        plsc.subcore_barrier()

        idx_wait(0)
        gissue(0, 0)

        def phase(c, p, guard_first):
            b, s = p % 2, p % 8
            bn, sn = (p + 1) % 2, (p + 1) % 8
            sprev = (p - 1) % 8

            if guard_first:
                @pl.when(c > 0)
                def _():
                    scw(bn, sprev)
            else:
                scw(bn, sprev)
            idx_issue(c + 5, (p + 5) % 8)
            idx_wait(sn)
            gissue(bn, sn)
            gwait(b, s)
            scatter_async(b, s)

        def octet(j, carry):
            c = 8 * j
            for p in range(8):
                phase(c + p, p, p == 0)
            return carry

        lax.fori_loop(0, (NCHUNK - 5) // 8, octet, 0)
        for c in range(NCHUNK - 5, NCHUNK):
            b, s = c % 2, c % 8
            bn, sn = (c + 1) % 2, (c + 1) % 8
            scw(bn, (c - 1) % 8)
            if c + 1 < NCHUNK:
                idx_wait(sn)
                gissue(bn, sn)
            gwait(b, s)
            scatter_async(b, s)
        scw((NCHUNK - 1) % 2, (NCHUNK - 1) % 8)
        plsc.subcore_barrier()
        _slab_out(acc, out_acc, cid, sid)
        if with_counts:
            @pl.when(sid < 15)
            def _():
                pltpu.sync_copy(cacc.at[pl.ds(sid * CP0, CP0)], cnt_v)
                pltpu.sync_copy(cnt_v,
                                out_cnt.at[pl.ds(cid * N + sid * CP0, CP0)])

            @pl.when(sid == 15)
            def _():
                pltpu.sync_copy(cacc.at[pl.ds(15 * CP0, CP1)],
                                cnt_v.at[pl.ds(0, CP1)])
                pltpu.sync_copy(cnt_v.at[pl.ds(0, CP1)],
                                out_cnt.at[pl.ds(cid * N + 15 * CP0, CP1)])

    return pl.kernel(body, mesh=mesh, out_type=out_type, scratch_types=scratch)


def _tc_pre(x, w1l, w1r, b1l):
    def body(x_ref, wl_ref, wr_ref, b_ref, t_ref, p_ref):
        xv = x_ref[...]
        t_ref[...] = jnp.dot(xv, wl_ref[...], preferred_element_type=jnp.float32)
        p_ref[...] = (jnp.dot(xv, wr_ref[...], preferred_element_type=jnp.float32)
                      + b_ref[...])

    return pl.pallas_call(
        body,
        out_shape=[jax.ShapeDtypeStruct((N, D), jnp.float32),
                   jax.ShapeDtypeStruct((N, D), jnp.float32)],
    )(x, w1l, w1r, b1l)


def _tc_mid(acc2n, cnt_a, cnt_b, p1, g1, be1, w2l, w2r, b2l):
    def body(a_ref, ca_ref, cb_ref, p_ref, g_ref, be_ref, wl_ref, wr_ref,
             b_ref, t_ref, p2_ref, cnt_ref):
        agg = a_ref[0:N, :] + a_ref[N:2 * N, :]
        cnt = jnp.maximum(ca_ref[...] + cb_ref[...], 1.0)
        s = agg / cnt + p_ref[...]
        mu = jnp.mean(s, axis=0, keepdims=True)
        var = jnp.mean((s - mu) ** 2, axis=0, keepdims=True)
        h = (s - mu) * lax.rsqrt(var + 1e-5) * g_ref[...] + be_ref[...]
        h = jnp.maximum(h, 0.0)
        t_ref[...] = jnp.dot(h, wl_ref[...], preferred_element_type=jnp.float32)
        p2_ref[...] = (jnp.dot(h, wr_ref[...], preferred_element_type=jnp.float32)
                       + b_ref[...])
        cnt_ref[...] = cnt

    return pl.pallas_call(
        body,
        out_shape=[jax.ShapeDtypeStruct((N, D), jnp.float32),
                   jax.ShapeDtypeStruct((N, D), jnp.float32),
                   jax.ShapeDtypeStruct((N, 1), jnp.float32)],
    )(acc2n, cnt_a, cnt_b, p1, g1, be1, w2l, w2r, b2l)


def _tc_post(acc2n, cnt, p2, g2, be2, batch2d, wlin_pad, blin_pad):
    def body(a_ref, c_ref, p_ref, g_ref, be_ref, bt_ref, wl_ref, bl_ref, o_ref):
        agg = a_ref[0:N, :] + a_ref[N:2 * N, :]
        s = agg / c_ref[...] + p_ref[...]
        mu = jnp.mean(s, axis=0, keepdims=True)
        var = jnp.mean((s - mu) ** 2, axis=0, keepdims=True)
        h = (s - mu) * lax.rsqrt(var + 1e-5) * g_ref[...] + be_ref[...]
        h = jnp.maximum(h, 0.0)
        gid = lax.broadcasted_iota(jnp.int32, (G, N), 0)
        oh = (gid == bt_ref[...]).astype(jnp.float32)
        sums = jnp.dot(oh, h, preferred_element_type=jnp.float32)
        cnts = jnp.maximum(jnp.sum(oh, axis=1, keepdims=True), 1.0)
        pooled = sums / cnts
        z = jnp.dot(pooled, wl_ref[...], preferred_element_type=jnp.float32) + bl_ref[...]
        o_ref[...] = 1.0 / (1.0 + jnp.exp(-z))

    return pl.pallas_call(
        body,
        out_shape=jax.ShapeDtypeStruct((G, D), jnp.float32),
    )(acc2n, cnt, p2, g2, be2, batch2d, wlin_pad, blin_pad)


def kernel(x, edge_index, batch, W1l, b1l, W1r, g1, be1, W2l, b2l, W2r, g2,
           be2, Wlin, blin):
    src = edge_index[0]
    dst = edge_index[1]
    zrow = jnp.zeros((CP0, D), jnp.float32)
    zcnt = jnp.zeros((CP0,), jnp.float32)
    ones = jnp.ones((K,), jnp.float32)

    t1, p1 = _tc_pre(x, W1l, W1r, b1l.reshape(1, D))

    acc1, cntp = _sc_agg(True)(t1, src, dst, zrow, zcnt, ones)

    t2, p2, cnt = _tc_mid(acc1, cntp[:N].reshape(N, 1),
                          cntp[N:].reshape(N, 1),
                          p1, g1.reshape(1, D), be1.reshape(1, D),
                          W2l, W2r, b2l.reshape(1, D))

    (acc2,) = _sc_agg(False)(t2, src, dst, zrow)

    wlin_pad = jnp.zeros((D, D), jnp.float32).at[:, :Wlin.shape[1]].set(Wlin)
    blin_pad = jnp.zeros((1, D), jnp.float32).at[0, :blin.shape[0]].set(blin)
    out = _tc_post(acc2, cnt, p2,
                   g2.reshape(1, D), be2.reshape(1, D),
                   batch.reshape(1, N).astype(jnp.int32), wlin_pad, blin_pad)
    return out[:, :Wlin.shape[1]]


# speedup vs baseline: 14.0676x; 1.1620x over previous
import functools

import jax
import jax.numpy as jnp
from jax import lax
from jax.experimental import pallas as pl
from jax.experimental.pallas import tpu as pltpu
from jax.experimental.pallas import tpu_sc as plsc

N = 10000
E = 320000
D = 128
G = 64
NC = 2
NS = 16
NW = NC * NS
EPW = E // NW
K = 80
NCHUNK = EPW // K
CP0 = 632
CP1 = N - 15 * CP0


def _slab_init(zsrc, accref, sid):
    @pl.when(sid < 15)
    def _():
        pltpu.sync_copy(zsrc, accref.at[pl.ds(sid * CP0, CP0)])

    @pl.when(sid == 15)
    def _():
        pltpu.sync_copy(zsrc.at[pl.ds(0, CP1)], accref.at[pl.ds(15 * CP0, CP1)])


def _slab_out(accref, outref, cid, sid):
    @pl.when(sid < 15)
    def _():
        pltpu.sync_copy(accref.at[pl.ds(sid * CP0, CP0)],
                        outref.at[pl.ds(cid * N + sid * CP0, CP0)])

    @pl.when(sid == 15)
    def _():
        pltpu.sync_copy(accref.at[pl.ds(15 * CP0, CP1)],
                        outref.at[pl.ds(cid * N + 15 * CP0, CP1)])


def _sc_agg(with_counts):
    mesh = plsc.VectorSubcoreMesh(core_axis_name="c", subcore_axis_name="s")
    out_type = [jax.ShapeDtypeStruct((NC * N, D), jnp.float32)]
    scratch = (
        [pltpu.VMEM((2, K), jnp.int32)] * 6
        + [pltpu.VMEM((K, D), jnp.float32) for _ in range(3)]
        + [pltpu.SemaphoreType.DMA] * 12
        + [pltpu.VMEM_SHARED((N, D), jnp.float32)]
    )
    if with_counts:
        out_type.append(jax.ShapeDtypeStruct((NC * N,), jnp.float32))
        scratch += [
            pltpu.VMEM((K,), jnp.float32),
            pltpu.VMEM_SHARED((N,), jnp.float32),
            pltpu.VMEM((CP0,), jnp.float32),
        ]

    def body(table, src, dst, zrow, *rest):
        if with_counts:
            zcnt, ones, out_acc, out_cnt = rest[0], rest[1], rest[2], rest[3]
            sc = rest[4:]
        else:
            out_acc = rest[0]
            sc = rest[1:]
        iset = sc[0:6]
        rows = sc[6:9]
        semg = sc[9:12]
        semsc = sc[12:15]
        semi = sc[15:21]
        acc = sc[21]
        if with_counts:
            ones_v, cacc, cnt_v = sc[22:25]

        cid = lax.axis_index("c")
        sid = lax.axis_index("s")
        base = (cid * NS + sid) * EPW

        def idx_issue(c, s):
            off = base + c * K
            pltpu.async_copy(src.at[pl.ds(off, K)], iset[s].at[0], semi[s])
            pltpu.async_copy(dst.at[pl.ds(off, K)], iset[s].at[1], semi[s])

        def idx_wait(s):
            pltpu.make_async_copy(src.at[pl.ds(0, K)], iset[s].at[0],
                                  semi[s]).wait()
            pltpu.make_async_copy(dst.at[pl.ds(0, K)], iset[s].at[1],
                                  semi[s]).wait()

        def gissue(b, s):
            pltpu.async_copy(table.at[iset[s].at[0]], rows[b], semg[b])

        def gwait(b, s):
            pltpu.make_async_copy(table.at[iset[s].at[0]], rows[b],
                                  semg[b]).wait()

        def scatter_async(b, s):
            pltpu.async_copy(rows[b], acc.at[iset[s].at[1]], semsc[b],
                             add=True)
            if with_counts:
                pltpu.async_copy(ones_v, cacc.at[iset[s].at[1]], semsc[b],
                                 add=True)

        def scw(b, s):
            pltpu.make_async_copy(rows[b], acc.at[iset[s].at[1]],
                                  semsc[b]).wait()
            if with_counts:
                pltpu.make_async_copy(ones_v, cacc.at[iset[s].at[1]],
                                      semsc[b]).wait()

        for p in range(5):
            idx_issue(p, p)
        _slab_init(zrow, acc, sid)
        if with_counts:
            pltpu.sync_copy(zcnt, cnt_v)

            @pl.when(sid < 15)
            def _():
                pltpu.sync_copy(cnt_v, cacc.at[pl.ds(sid * CP0, CP0)])

            @pl.when(sid == 15)
            def _():
                pltpu.sync_copy(cnt_v.at[pl.ds(0, CP1)],
                                cacc.at[pl.ds(15 * CP0, CP1)])

            pltpu.sync_copy(ones, ones_v)
        plsc.subcore_barrier()

        idx_wait(0)
        gissue(0, 0)
        idx_wait(1)
        gissue(1, 1)

        def phase(c, p, guard_first):
            b, s = p % 3, p % 6
            b2, s2 = (p + 2) % 3, (p + 2) % 6
            bprev, sprev = (p - 1) % 3, (p - 1) % 6

            if guard_first:
                @pl.when(c > 0)
                def _():
                    scw(bprev, sprev)
            else:
                scw(bprev, sprev)
            idx_issue(c + 5, (p + 5) % 6)
            idx_wait(s2)
            gissue(b2, s2)
            gwait(b, s)
            scatter_async(b, s)

        def sextet(j, carry):
            c = 6 * j
            for p in range(6):
                phase(c + p, p, p == 0)
            return carry

        lax.fori_loop(0, (NCHUNK - 5) // 6, sextet, 0)
        for c in range(NCHUNK - 5, NCHUNK):
            b, s = c % 3, c % 6
            b2, s2 = (c + 2) % 3, (c + 2) % 6
            scw((c - 1) % 3, (c - 1) % 6)
            if c + 2 < NCHUNK:
                idx_wait(s2)
                gissue(b2, s2)
            gwait(b, s)
            scatter_async(b, s)
        scw((NCHUNK - 1) % 3, (NCHUNK - 1) % 6)
        plsc.subcore_barrier()
        _slab_out(acc, out_acc, cid, sid)
        if with_counts:
            @pl.when(sid < 15)
            def _():
                pltpu.sync_copy(cacc.at[pl.ds(sid * CP0, CP0)], cnt_v)
                pltpu.sync_copy(cnt_v,
                                out_cnt.at[pl.ds(cid * N + sid * CP0, CP0)])

            @pl.when(sid == 15)
            def _():
                pltpu.sync_copy(cacc.at[pl.ds(15 * CP0, CP1)],
                                cnt_v.at[pl.ds(0, CP1)])
                pltpu.sync_copy(cnt_v.at[pl.ds(0, CP1)],
                                out_cnt.at[pl.ds(cid * N + 15 * CP0, CP1)])

    return pl.kernel(body, mesh=mesh, out_type=out_type, scratch_types=scratch)


def _tc_pre(x, w1l, w1r, b1l):
    def body(x_ref, wl_ref, wr_ref, b_ref, t_ref, p_ref):
        xv = x_ref[...]
        t_ref[...] = jnp.dot(xv, wl_ref[...], preferred_element_type=jnp.float32)
        p_ref[...] = (jnp.dot(xv, wr_ref[...], preferred_element_type=jnp.float32)
                      + b_ref[...])

    return pl.pallas_call(
        body,
        out_shape=[jax.ShapeDtypeStruct((N, D), jnp.float32),
                   jax.ShapeDtypeStruct((N, D), jnp.float32)],
    )(x, w1l, w1r, b1l)


def _tc_mid(acc2n, cnt_a, cnt_b, p1, g1, be1, w2l, w2r, b2l):
    def body(a_ref, ca_ref, cb_ref, p_ref, g_ref, be_ref, wl_ref, wr_ref,
             b_ref, t_ref, p2_ref, cnt_ref):
        agg = a_ref[0:N, :] + a_ref[N:2 * N, :]
        cnt = jnp.maximum(ca_ref[...] + cb_ref[...], 1.0)
        s = agg / cnt + p_ref[...]
        mu = jnp.mean(s, axis=0, keepdims=True)
        var = jnp.mean((s - mu) ** 2, axis=0, keepdims=True)
        h = (s - mu) * lax.rsqrt(var + 1e-5) * g_ref[...] + be_ref[...]
        h = jnp.maximum(h, 0.0)
        t_ref[...] = jnp.dot(h, wl_ref[...], preferred_element_type=jnp.float32)
        p2_ref[...] = (jnp.dot(h, wr_ref[...], preferred_element_type=jnp.float32)
                       + b_ref[...])
        cnt_ref[...] = cnt

    return pl.pallas_call(
        body,
        out_shape=[jax.ShapeDtypeStruct((N, D), jnp.float32),
                   jax.ShapeDtypeStruct((N, D), jnp.float32),
                   jax.ShapeDtypeStruct((N, 1), jnp.float32)],
    )(acc2n, cnt_a, cnt_b, p1, g1, be1, w2l, w2r, b2l)


def _tc_post(acc2n, cnt, p2, g2, be2, batch2d, wlin_pad, blin_pad):
    def body(a_ref, c_ref, p_ref, g_ref, be_ref, bt_ref, wl_ref, bl_ref, o_ref):
        agg = a_ref[0:N, :] + a_ref[N:2 * N, :]
        s = agg / c_ref[...] + p_ref[...]
        mu = jnp.mean(s, axis=0, keepdims=True)
        var = jnp.mean((s - mu) ** 2, axis=0, keepdims=True)
        h = (s - mu) * lax.rsqrt(var + 1e-5) * g_ref[...] + be_ref[...]
        h = jnp.maximum(h, 0.0)
        gid = lax.broadcasted_iota(jnp.int32, (G, N), 0)
        oh = (gid == bt_ref[...]).astype(jnp.float32)
        sums = jnp.dot(oh, h, preferred_element_type=jnp.float32)
        cnts = jnp.maximum(jnp.sum(oh, axis=1, keepdims=True), 1.0)
        pooled = sums / cnts
        z = jnp.dot(pooled, wl_ref[...], preferred_element_type=jnp.float32) + bl_ref[...]
        o_ref[...] = 1.0 / (1.0 + jnp.exp(-z))

    return pl.pallas_call(
        body,
        out_shape=jax.ShapeDtypeStruct((G, D), jnp.float32),
    )(acc2n, cnt, p2, g2, be2, batch2d, wlin_pad, blin_pad)


def kernel(x, edge_index, batch, W1l, b1l, W1r, g1, be1, W2l, b2l, W2r, g2,
           be2, Wlin, blin):
    src = edge_index[0]
    dst = edge_index[1]
    zrow = jnp.zeros((CP0, D), jnp.float32)
    zcnt = jnp.zeros((CP0,), jnp.float32)
    ones = jnp.ones((K,), jnp.float32)

    t1, p1 = _tc_pre(x, W1l, W1r, b1l.reshape(1, D))

    acc1, cntp = _sc_agg(True)(t1, src, dst, zrow, zcnt, ones)

    t2, p2, cnt = _tc_mid(acc1, cntp[:N].reshape(N, 1),
                          cntp[N:].reshape(N, 1),
                          p1, g1.reshape(1, D), be1.reshape(1, D),
                          W2l, W2r, b2l.reshape(1, D))

    (acc2,) = _sc_agg(False)(t2, src, dst, zrow)

    wlin_pad = jnp.zeros((D, D), jnp.float32).at[:, :Wlin.shape[1]].set(Wlin)
    blin_pad = jnp.zeros((1, D), jnp.float32).at[0, :blin.shape[0]].set(blin)
    out = _tc_post(acc2, cnt, p2,
                   g2.reshape(1, D), be2.reshape(1, D),
                   batch.reshape(1, N).astype(jnp.int32), wlin_pad, blin_pad)
    return out[:, :Wlin.shape[1]]
